# baseline (speedup 1.0000x reference)
import functools

import jax
import jax.numpy as jnp
from jax import lax
from jax.experimental import pallas as pl
from jax.experimental.pallas import tpu as pltpu
from jax.experimental.pallas import tpu_sc as plsc

QK = 96
NWIN = 1024
HALF = NWIN // 2
K = 4
SCALE = QK ** (-0.5)
NW = 32


def _half_top4(x, col):
    vals, idxs = [], []
    for j in range(K):
        m = jnp.max(x, axis=-1, keepdims=True)
        am = jnp.min(jnp.where(x == m, col, float(NWIN)),
                     axis=-1, keepdims=True)
        vals.append(m)
        idxs.append(am)
        if j < K - 1:
            x = jnp.where(col == am, -jnp.inf, x)
    return vals, idxs


def _tc_body(g_ref, wq_ref, bq_ref, wk_ref, bk_ref, cv_ref, ci_ref):
    g = g_ref[0]
    q = lax.dot_general(g, wq_ref[...], (((1,), (1,)), ((), ())),
                        preferred_element_type=jnp.float32) + bq_ref[0]
    k = lax.dot_general(g, wk_ref[...], (((1,), (1,)), ((), ())),
                        preferred_element_type=jnp.float32) + bk_ref[0]
    attn = lax.dot_general(q * SCALE, k, (((1,), (1,)), ((), ())),
                           preferred_element_type=jnp.float32)
    col = lax.broadcasted_iota(
        jnp.int32, (NWIN, HALF), 1).astype(jnp.float32)
    v1, i1 = _half_top4(attn[:, :HALF], col)
    v2, i2 = _half_top4(attn[:, HALF:], col + float(HALF))
    cv_ref[0] = jnp.concatenate(v1 + v2, axis=-1)
    ci_ref[0] = jnp.concatenate(i1 + i2, axis=-1).astype(jnp.int32)


def _tc_candidates(g_win, Wq, bq, Wk, bk):
    B = g_win.shape[0]
    grid_spec = pl.GridSpec(
        grid=(B,),
        in_specs=[
            pl.BlockSpec((1, NWIN, QK), lambda b: (b, 0, 0)),
            pl.BlockSpec((QK, QK), lambda b: (0, 0)),
            pl.BlockSpec((1, QK), lambda b: (0, 0)),
            pl.BlockSpec((QK, QK), lambda b: (0, 0)),
            pl.BlockSpec((1, QK), lambda b: (0, 0)),
        ],
        out_specs=[
            pl.BlockSpec((1, NWIN, 2 * K), lambda b: (b, 0, 0)),
            pl.BlockSpec((1, NWIN, 2 * K), lambda b: (b, 0, 0)),
        ],
    )
    return pl.pallas_call(
        _tc_body,
        grid_spec=grid_spec,
        out_shape=[
            jax.ShapeDtypeStruct((B, NWIN, 2 * K), jnp.float32),
            jax.ShapeDtypeStruct((B, NWIN, 2 * K), jnp.int32),
        ],
    )(g_win, Wq, bq.reshape(1, QK), Wk, bk.reshape(1, QK))


def _take16(x, idx):
    dnums = lax.GatherDimensionNumbers(
        offset_dims=(), collapsed_slice_dims=(0,), start_index_map=(0,))
    return lax.gather(x, idx[:, None], dnums, slice_sizes=(1,),
                      mode=lax.GatherScatterMode.PROMISE_IN_BOUNDS)


def _sort8_desc(v, i, lanes):
    l7 = lanes & 7
    for d in (7, 2, 1):
        p = (lanes & ~7) | (l7 ^ d)
        pv = _take16(v, p)
        pi = _take16(i, p)
        take = (v > pv) | ((v == pv) & (i < pi))
        inv = lanes > p
        keep = take != inv
        v = jnp.where(keep, v, pv)
        i = jnp.where(keep, i, pi)
    return v, i


def _softmax8(v, lanes):
    m = _take16(v, lanes & 8)
    e = jnp.exp(v - m)
    s = e + _take16(e, lanes ^ 1)
    s = s + _take16(s, lanes ^ 2)
    return e / s


def _sc_merge(cv2d, ci2d):
    mesh = plsc.VectorSubcoreMesh(core_axis_name="c", subcore_axis_name="s")
    info = plsc.get_sparse_core_info()
    nc = info.num_cores
    n_in = cv2d.shape[1]
    n_out = n_in // 2

    @functools.partial(
        pl.kernel,
        mesh=mesh,
        out_type=[
            jax.ShapeDtypeStruct((NW, n_out), jnp.float32),
            jax.ShapeDtypeStruct((NW, n_out), jnp.int32),
        ],
        scratch_types=[
            pltpu.VMEM((n_in,), jnp.float32),
            pltpu.VMEM((n_in,), jnp.int32),
            pltpu.VMEM((n_out,), jnp.float32),
            pltpu.VMEM((n_out,), jnp.int32),
        ],
    )
    def sc_kernel(cv_hbm, ci_hbm, w_hbm, ix_hbm, vbuf, ibuf, wbuf, obuf):
        w = lax.axis_index("s") * nc + lax.axis_index("c")
        pltpu.sync_copy(cv_hbm.at[w], vbuf)
        pltpu.sync_copy(ci_hbm.at[w], ibuf)
        lanes = lax.iota(jnp.int32, 16)
        pk = (lanes & 3) + ((lanes & 4) << 1)
        low8 = lanes < 8

        def body(t, _):
            base = t * 32
            out_w = []
            out_i = []
            for hv in (0, 16):
                v = vbuf[pl.ds(base + hv, 16)]
                i = ibuf[pl.ds(base + hv, 16)]
                v, i = _sort8_desc(v, i, lanes)
                sw = _softmax8(v, lanes)
                out_w.append(_take16(sw, pk))
                out_i.append(_take16(i, pk))
            wbuf[pl.ds(t * 16, 16)] = jnp.where(low8, out_w[0], out_w[1])
            obuf[pl.ds(t * 16, 16)] = jnp.where(low8, out_i[0], out_i[1])
            return 0

        lax.fori_loop(0, n_in // 32, body, 0)
        pltpu.sync_copy(wbuf, w_hbm.at[w])
        pltpu.sync_copy(obuf, ix_hbm.at[w])

    return sc_kernel(cv2d, ci2d)


CHUNK = 16


@jax.jit
def kernel(g_win, Wq, bq, Wk, bk):
    B = g_win.shape[0]
    ws, ixs = [], []
    for c in range(0, B, CHUNK):
        g_c = lax.slice_in_dim(g_win, c, c + CHUNK, axis=0)
        cv, ci = _tc_candidates(g_c, Wq, bq, Wk, bk)
        w2d, i2d = _sc_merge(cv.reshape(NW, CHUNK * NWIN * 2 * K // NW),
                             ci.reshape(NW, CHUNK * NWIN * 2 * K // NW))
        ws.append(w2d.reshape(CHUNK, NWIN, K))
        ixs.append(i2d.reshape(CHUNK, NWIN, K))
    return (jnp.concatenate(ws, axis=0), jnp.concatenate(ixs, axis=0))


# speedup vs baseline: 1.3297x; 1.3297x over previous
import functools

import jax
import jax.numpy as jnp
from jax import lax
from jax.experimental import pallas as pl
from jax.experimental.pallas import tpu as pltpu
from jax.experimental.pallas import tpu_sc as plsc

QK = 96
NWIN = 1024
HALF = NWIN // 2
K = 4
SCALE = QK ** (-0.5)
NW = 32


def _half_top4(x, col):
    vals, idxs = [], []
    for j in range(K):
        m = jnp.max(x, axis=-1, keepdims=True)
        am = jnp.min(jnp.where(x == m, col, float(NWIN)),
                     axis=-1, keepdims=True)
        vals.append(m)
        idxs.append(am)
        if j < K - 1:
            x = jnp.where(col == am, -jnp.inf, x)
    return vals, idxs


def _tc_body(g_ref, wq_ref, bq_ref, wk_ref, bk_ref, cv_ref, ci_ref):
    g = g_ref[0]
    q = lax.dot_general(g, wq_ref[...], (((1,), (1,)), ((), ())),
                        preferred_element_type=jnp.float32) + bq_ref[0]
    k = lax.dot_general(g, wk_ref[...], (((1,), (1,)), ((), ())),
                        preferred_element_type=jnp.float32) + bk_ref[0]
    attn = lax.dot_general(q * SCALE, k, (((1,), (1,)), ((), ())),
                           preferred_element_type=jnp.float32)
    col = lax.broadcasted_iota(
        jnp.int32, (NWIN, HALF), 1).astype(jnp.float32)
    v1, i1 = _half_top4(attn[:, :HALF], col)
    v2, i2 = _half_top4(attn[:, HALF:], col + float(HALF))
    cv_ref[0] = jnp.concatenate(v1 + v2, axis=-1)
    ci_ref[0] = jnp.concatenate(i1 + i2, axis=-1).astype(jnp.int32)


def _tc_candidates(g_win, Wq, bq, Wk, bk):
    B = g_win.shape[0]
    grid_spec = pl.GridSpec(
        grid=(B,),
        in_specs=[
            pl.BlockSpec((1, NWIN, QK), lambda b: (b, 0, 0)),
            pl.BlockSpec((QK, QK), lambda b: (0, 0)),
            pl.BlockSpec((1, QK), lambda b: (0, 0)),
            pl.BlockSpec((QK, QK), lambda b: (0, 0)),
            pl.BlockSpec((1, QK), lambda b: (0, 0)),
        ],
        out_specs=[
            pl.BlockSpec((1, NWIN, 2 * K), lambda b: (b, 0, 0)),
            pl.BlockSpec((1, NWIN, 2 * K), lambda b: (b, 0, 0)),
        ],
    )
    return pl.pallas_call(
        _tc_body,
        grid_spec=grid_spec,
        out_shape=[
            jax.ShapeDtypeStruct((B, NWIN, 2 * K), jnp.float32),
            jax.ShapeDtypeStruct((B, NWIN, 2 * K), jnp.int32),
        ],
    )(g_win, Wq, bq.reshape(1, QK), Wk, bk.reshape(1, QK))


def _take16(x, idx):
    dnums = lax.GatherDimensionNumbers(
        offset_dims=(), collapsed_slice_dims=(0,), start_index_map=(0,))
    return lax.gather(x, idx[:, None], dnums, slice_sizes=(1,),
                      mode=lax.GatherScatterMode.PROMISE_IN_BOUNDS)


def _sort8_desc(v, i, lanes):
    l7 = lanes & 7
    for d in (7, 2, 1):
        p = (lanes & ~7) | (l7 ^ d)
        pv = _take16(v, p)
        pi = _take16(i, p)
        take = (v > pv) | ((v == pv) & (i < pi))
        inv = lanes > p
        keep = take != inv
        v = jnp.where(keep, v, pv)
        i = jnp.where(keep, i, pi)
    return v, i


def _softmax8(v, lanes):
    m = _take16(v, lanes & 8)
    e = jnp.exp(v - m)
    s = e + _take16(e, lanes ^ 1)
    s = s + _take16(s, lanes ^ 2)
    return e / s


def _sc_merge(cv2d, ci2d):
    mesh = plsc.VectorSubcoreMesh(core_axis_name="c", subcore_axis_name="s")
    info = plsc.get_sparse_core_info()
    nc = info.num_cores
    n_in = cv2d.shape[1]
    n_out = n_in // 2

    @functools.partial(
        pl.kernel,
        mesh=mesh,
        out_type=[
            jax.ShapeDtypeStruct((NW, n_out), jnp.float32),
            jax.ShapeDtypeStruct((NW, n_out), jnp.int32),
        ],
        scratch_types=[
            pltpu.VMEM((n_in,), jnp.float32),
            pltpu.VMEM((n_in,), jnp.int32),
            pltpu.VMEM((n_out,), jnp.float32),
            pltpu.VMEM((n_out,), jnp.int32),
        ],
    )
    def sc_kernel(cv_hbm, ci_hbm, w_hbm, ix_hbm, vbuf, ibuf, wbuf, obuf):
        w = lax.axis_index("s") * nc + lax.axis_index("c")
        pltpu.sync_copy(cv_hbm.at[w], vbuf)
        pltpu.sync_copy(ci_hbm.at[w], ibuf)
        lanes = lax.iota(jnp.int32, 16)
        pk = (lanes & 3) + ((lanes & 4) << 1)
        low8 = lanes < 8

        def body(t, _):
            base = t * 32
            out_w = []
            out_i = []
            for hv in (0, 16):
                v = vbuf[pl.ds(base + hv, 16)]
                i = ibuf[pl.ds(base + hv, 16)]
                v, i = _sort8_desc(v, i, lanes)
                sw = _softmax8(v, lanes)
                out_w.append(_take16(sw, pk))
                out_i.append(_take16(i, pk))
            wbuf[pl.ds(t * 16, 16)] = jnp.where(low8, out_w[0], out_w[1])
            obuf[pl.ds(t * 16, 16)] = jnp.where(low8, out_i[0], out_i[1])
            return 0

        lax.fori_loop(0, n_in // 32, body, 0)
        pltpu.sync_copy(wbuf, w_hbm.at[w])
        pltpu.sync_copy(obuf, ix_hbm.at[w])

    return sc_kernel(cv2d, ci2d)


CHUNK = 32


@jax.jit
def kernel(g_win, Wq, bq, Wk, bk):
    B = g_win.shape[0]
    ws, ixs = [], []
    for c in range(0, B, CHUNK):
        g_c = lax.slice_in_dim(g_win, c, c + CHUNK, axis=0)
        cv, ci = _tc_candidates(g_c, Wq, bq, Wk, bk)
        w2d, i2d = _sc_merge(cv.reshape(NW, CHUNK * NWIN * 2 * K // NW),
                             ci.reshape(NW, CHUNK * NWIN * 2 * K // NW))
        ws.append(w2d.reshape(CHUNK, NWIN, K))
        ixs.append(i2d.reshape(CHUNK, NWIN, K))
    return (jnp.concatenate(ws, axis=0), jnp.concatenate(ixs, axis=0))


# speedup vs baseline: 1.3534x; 1.0178x over previous
import functools

import jax
import jax.numpy as jnp
from jax import lax
from jax.experimental import pallas as pl
from jax.experimental.pallas import tpu as pltpu
from jax.experimental.pallas import tpu_sc as plsc

QK = 96
NWIN = 1024
HALF = NWIN // 2
K = 4
SCALE = QK ** (-0.5)
NW = 32


def _half_top4(x, col):
    vals, idxs = [], []
    for j in range(K):
        m = jnp.max(x, axis=-1, keepdims=True)
        am = jnp.min(jnp.where(x == m, col, float(NWIN)),
                     axis=-1, keepdims=True)
        vals.append(m)
        idxs.append(am)
        if j < K - 1:
            x = jnp.where(col == am, -jnp.inf, x)
    return vals, idxs


def _tc_body(g_ref, wq_ref, bq_ref, wk_ref, bk_ref, cv_ref, ci_ref):
    g = g_ref[0]
    q = lax.dot_general(g, wq_ref[...], (((1,), (1,)), ((), ())),
                        preferred_element_type=jnp.float32) + bq_ref[0]
    k = lax.dot_general(g, wk_ref[...], (((1,), (1,)), ((), ())),
                        preferred_element_type=jnp.float32) + bk_ref[0]
    attn = lax.dot_general(q * SCALE, k, (((1,), (1,)), ((), ())),
                           preferred_element_type=jnp.float32)
    col = lax.broadcasted_iota(
        jnp.int32, (NWIN, HALF), 1).astype(jnp.float32)
    v1, i1 = _half_top4(attn[:, :HALF], col)
    v2, i2 = _half_top4(attn[:, HALF:], col + float(HALF))
    cv_ref[0] = jnp.concatenate(v1 + v2, axis=-1)
    ci_ref[0] = jnp.concatenate(i1 + i2, axis=-1).astype(jnp.int32)


def _tc_candidates(g_win, Wq, bq, Wk, bk):
    B = g_win.shape[0]
    grid_spec = pl.GridSpec(
        grid=(B,),
        in_specs=[
            pl.BlockSpec((1, NWIN, QK), lambda b: (b, 0, 0)),
            pl.BlockSpec((QK, QK), lambda b: (0, 0)),
            pl.BlockSpec((1, QK), lambda b: (0, 0)),
            pl.BlockSpec((QK, QK), lambda b: (0, 0)),
            pl.BlockSpec((1, QK), lambda b: (0, 0)),
        ],
        out_specs=[
            pl.BlockSpec((1, NWIN, 2 * K), lambda b: (b, 0, 0)),
            pl.BlockSpec((1, NWIN, 2 * K), lambda b: (b, 0, 0)),
        ],
    )
    return pl.pallas_call(
        _tc_body,
        grid_spec=grid_spec,
        out_shape=[
            jax.ShapeDtypeStruct((B, NWIN, 2 * K), jnp.float32),
            jax.ShapeDtypeStruct((B, NWIN, 2 * K), jnp.int32),
        ],
    )(g_win, Wq, bq.reshape(1, QK), Wk, bk.reshape(1, QK))


def _take16(x, idx):
    dnums = lax.GatherDimensionNumbers(
        offset_dims=(), collapsed_slice_dims=(0,), start_index_map=(0,))
    return lax.gather(x, idx[:, None], dnums, slice_sizes=(1,),
                      mode=lax.GatherScatterMode.PROMISE_IN_BOUNDS)


def _sort8_desc(v, i, lanes):
    l7 = lanes & 7
    for d in (7, 2, 1):
        p = (lanes & ~7) | (l7 ^ d)
        pv = _take16(v, p)
        pi = _take16(i, p)
        take = (v > pv) | ((v == pv) & (i < pi))
        inv = lanes > p
        keep = take != inv
        v = jnp.where(keep, v, pv)
        i = jnp.where(keep, i, pi)
    return v, i


def _softmax8(v, lanes):
    m = _take16(v, lanes & 8)
    e = jnp.exp(v - m)
    s = e + _take16(e, lanes ^ 1)
    s = s + _take16(s, lanes ^ 2)
    return e / s


def _sc_merge(cv2d, ci2d):
    mesh = plsc.VectorSubcoreMesh(core_axis_name="c", subcore_axis_name="s")
    info = plsc.get_sparse_core_info()
    nc = info.num_cores
    n_in = cv2d.shape[1]
    n_out = n_in // 2

    @functools.partial(
        pl.kernel,
        mesh=mesh,
        out_type=[
            jax.ShapeDtypeStruct((NW, n_out), jnp.float32),
            jax.ShapeDtypeStruct((NW, n_out), jnp.int32),
        ],
        scratch_types=[
            pltpu.VMEM((n_in,), jnp.float32),
            pltpu.VMEM((n_in,), jnp.int32),
            pltpu.VMEM((n_out,), jnp.float32),
            pltpu.VMEM((n_out,), jnp.int32),
        ],
    )
    def sc_kernel(cv_hbm, ci_hbm, w_hbm, ix_hbm, vbuf, ibuf, wbuf, obuf):
        w = lax.axis_index("s") * nc + lax.axis_index("c")
        pltpu.sync_copy(cv_hbm.at[w], vbuf)
        pltpu.sync_copy(ci_hbm.at[w], ibuf)
        lanes = lax.iota(jnp.int32, 16)
        pk = (lanes & 3) + ((lanes & 4) << 1)
        low8 = lanes < 8

        def body(t, _):
            base = t * 64
            out_w = []
            out_i = []
            for hv in (0, 16, 32, 48):
                v = vbuf[pl.ds(base + hv, 16)]
                i = ibuf[pl.ds(base + hv, 16)]
                v, i = _sort8_desc(v, i, lanes)
                sw = _softmax8(v, lanes)
                out_w.append(_take16(sw, pk))
                out_i.append(_take16(i, pk))
            for g in (0, 1):
                wbuf[pl.ds(t * 32 + g * 16, 16)] = jnp.where(
                    low8, out_w[2 * g], out_w[2 * g + 1])
                obuf[pl.ds(t * 32 + g * 16, 16)] = jnp.where(
                    low8, out_i[2 * g], out_i[2 * g + 1])
            return 0

        lax.fori_loop(0, n_in // 64, body, 0)
        pltpu.sync_copy(wbuf, w_hbm.at[w])
        pltpu.sync_copy(obuf, ix_hbm.at[w])

    return sc_kernel(cv2d, ci2d)


CHUNK = 32


@jax.jit
def kernel(g_win, Wq, bq, Wk, bk):
    B = g_win.shape[0]
    ws, ixs = [], []
    for c in range(0, B, CHUNK):
        g_c = lax.slice_in_dim(g_win, c, c + CHUNK, axis=0)
        cv, ci = _tc_candidates(g_c, Wq, bq, Wk, bk)
        w2d, i2d = _sc_merge(cv.reshape(NW, CHUNK * NWIN * 2 * K // NW),
                             ci.reshape(NW, CHUNK * NWIN * 2 * K // NW))
        ws.append(w2d.reshape(CHUNK, NWIN, K))
        ixs.append(i2d.reshape(CHUNK, NWIN, K))
    return (jnp.concatenate(ws, axis=0), jnp.concatenate(ixs, axis=0))


# speedup vs baseline: 1.3535x; 1.0001x over previous
import functools

import jax
import jax.numpy as jnp
from jax import lax
from jax.experimental import pallas as pl
from jax.experimental.pallas import tpu as pltpu
from jax.experimental.pallas import tpu_sc as plsc

QK = 96
NWIN = 1024
HALF = NWIN // 2
K = 4
SCALE = QK ** (-0.5)
NW = 32


def _half_top4(x, col):
    vals, idxs = [], []
    for j in range(K):
        m = jnp.max(x, axis=-1, keepdims=True)
        am = jnp.min(jnp.where(x == m, col, float(NWIN)),
                     axis=-1, keepdims=True)
        vals.append(m)
        idxs.append(am)
        if j < K - 1:
            x = jnp.where(col == am, -jnp.inf, x)
    return vals, idxs


def _tc_body(g_ref, wq_ref, bq_ref, wk_ref, bk_ref, cv_ref, ci_ref):
    g = g_ref[0]
    q = lax.dot_general(g, wq_ref[...], (((1,), (1,)), ((), ())),
                        preferred_element_type=jnp.float32) + bq_ref[0]
    k = lax.dot_general(g, wk_ref[...], (((1,), (1,)), ((), ())),
                        preferred_element_type=jnp.float32) + bk_ref[0]
    attn = lax.dot_general(q * SCALE, k, (((1,), (1,)), ((), ())),
                           preferred_element_type=jnp.float32)
    col = lax.broadcasted_iota(
        jnp.int32, (NWIN, HALF), 1).astype(jnp.float32)
    v1, i1 = _half_top4(attn[:, :HALF], col)
    v2, i2 = _half_top4(attn[:, HALF:], col + float(HALF))
    cv_ref[0] = jnp.concatenate(v1 + v2, axis=-1)
    ci_ref[0] = jnp.concatenate(i1 + i2, axis=-1).astype(jnp.int32)


def _tc_candidates(g_win, Wq, bq, Wk, bk):
    B = g_win.shape[0]
    grid_spec = pl.GridSpec(
        grid=(B,),
        in_specs=[
            pl.BlockSpec((1, NWIN, QK), lambda b: (b, 0, 0)),
            pl.BlockSpec((QK, QK), lambda b: (0, 0)),
            pl.BlockSpec((1, QK), lambda b: (0, 0)),
            pl.BlockSpec((QK, QK), lambda b: (0, 0)),
            pl.BlockSpec((1, QK), lambda b: (0, 0)),
        ],
        out_specs=[
            pl.BlockSpec((1, NWIN, 2 * K), lambda b: (b, 0, 0)),
            pl.BlockSpec((1, NWIN, 2 * K), lambda b: (b, 0, 0)),
        ],
    )
    return pl.pallas_call(
        _tc_body,
        grid_spec=grid_spec,
        out_shape=[
            jax.ShapeDtypeStruct((B, NWIN, 2 * K), jnp.float32),
            jax.ShapeDtypeStruct((B, NWIN, 2 * K), jnp.int32),
        ],
    )(g_win, Wq, bq.reshape(1, QK), Wk, bk.reshape(1, QK))


def _take16(x, idx):
    dnums = lax.GatherDimensionNumbers(
        offset_dims=(), collapsed_slice_dims=(0,), start_index_map=(0,))
    return lax.gather(x, idx[:, None], dnums, slice_sizes=(1,),
                      mode=lax.GatherScatterMode.PROMISE_IN_BOUNDS)


def _sort8_desc(v, i, lanes):
    l7 = lanes & 7
    for d in (7, 2, 1):
        p = (lanes & ~7) | (l7 ^ d)
        pv = _take16(v, p)
        pi = _take16(i, p)
        take = (v > pv) | ((v == pv) & (i < pi))
        inv = lanes > p
        keep = take != inv
        v = jnp.where(keep, v, pv)
        i = jnp.where(keep, i, pi)
    return v, i


def _softmax8(v, lanes):
    m = _take16(v, lanes & 8)
    e = jnp.exp(v - m)
    s = e + _take16(e, lanes ^ 1)
    s = s + _take16(s, lanes ^ 2)
    return e / s


def _sc_merge(cv2d, ci2d):
    mesh = plsc.VectorSubcoreMesh(core_axis_name="c", subcore_axis_name="s")
    info = plsc.get_sparse_core_info()
    nc = info.num_cores
    n_in = cv2d.shape[1]
    n_out = n_in // 2

    @functools.partial(
        pl.kernel,
        mesh=mesh,
        out_type=[
            jax.ShapeDtypeStruct((NW, n_out), jnp.float32),
            jax.ShapeDtypeStruct((NW, n_out), jnp.int32),
        ],
        scratch_types=[
            pltpu.VMEM((n_in,), jnp.float32),
            pltpu.VMEM((n_in,), jnp.int32),
            pltpu.VMEM((n_out,), jnp.float32),
            pltpu.VMEM((n_out,), jnp.int32),
        ],
    )
    def sc_kernel(cv_hbm, ci_hbm, w_hbm, ix_hbm, vbuf, ibuf, wbuf, obuf):
        w = lax.axis_index("s") * nc + lax.axis_index("c")
        pltpu.sync_copy(cv_hbm.at[w], vbuf)
        pltpu.sync_copy(ci_hbm.at[w], ibuf)
        lanes = lax.iota(jnp.int32, 16)
        pk = (lanes & 3) + ((lanes & 4) << 1)
        low8 = lanes < 8

        def body(t, _):
            base = t * 128
            out_w = []
            out_i = []
            for hv in range(0, 128, 16):
                v = vbuf[pl.ds(base + hv, 16)]
                i = ibuf[pl.ds(base + hv, 16)]
                v, i = _sort8_desc(v, i, lanes)
                sw = _softmax8(v, lanes)
                out_w.append(_take16(sw, pk))
                out_i.append(_take16(i, pk))
            for g in range(4):
                wbuf[pl.ds(t * 64 + g * 16, 16)] = jnp.where(
                    low8, out_w[2 * g], out_w[2 * g + 1])
                obuf[pl.ds(t * 64 + g * 16, 16)] = jnp.where(
                    low8, out_i[2 * g], out_i[2 * g + 1])
            return 0

        lax.fori_loop(0, n_in // 128, body, 0)
        pltpu.sync_copy(wbuf, w_hbm.at[w])
        pltpu.sync_copy(obuf, ix_hbm.at[w])

    return sc_kernel(cv2d, ci2d)


CHUNK = 32


@jax.jit
def kernel(g_win, Wq, bq, Wk, bk):
    B = g_win.shape[0]
    ws, ixs = [], []
    for c in range(0, B, CHUNK):
        g_c = lax.slice_in_dim(g_win, c, c + CHUNK, axis=0)
        cv, ci = _tc_candidates(g_c, Wq, bq, Wk, bk)
        w2d, i2d = _sc_merge(cv.reshape(NW, CHUNK * NWIN * 2 * K // NW),
                             ci.reshape(NW, CHUNK * NWIN * 2 * K // NW))
        ws.append(w2d.reshape(CHUNK, NWIN, K))
        ixs.append(i2d.reshape(CHUNK, NWIN, K))
    return (jnp.concatenate(ws, axis=0), jnp.concatenate(ixs, axis=0))


# speedup vs baseline: 1.3558x; 1.0017x over previous
import functools

import jax
import jax.numpy as jnp
from jax import lax
from jax.experimental import pallas as pl
from jax.experimental.pallas import tpu as pltpu
from jax.experimental.pallas import tpu_sc as plsc

QK = 96
NWIN = 1024
HALF = NWIN // 2
K = 4
SCALE = QK ** (-0.5)
NW = 32


def _half_top4(x, col):
    vals, idxs = [], []
    for j in range(K):
        m = jnp.max(x, axis=-1, keepdims=True)
        am = jnp.min(jnp.where(x == m, col, float(NWIN)),
                     axis=-1, keepdims=True)
        vals.append(m)
        idxs.append(am)
        if j < K - 1:
            x = jnp.where(col == am, -jnp.inf, x)
    return vals, idxs


def _tc_body(g_ref, wq_ref, bq_ref, wk_ref, bk_ref, cv_ref, ci_ref):
    g = g_ref[0]
    q = lax.dot_general(g, wq_ref[...], (((1,), (1,)), ((), ())),
                        preferred_element_type=jnp.float32) + bq_ref[0]
    k = lax.dot_general(g, wk_ref[...], (((1,), (1,)), ((), ())),
                        preferred_element_type=jnp.float32) + bk_ref[0]
    attn = lax.dot_general(q * SCALE, k, (((1,), (1,)), ((), ())),
                           preferred_element_type=jnp.float32)
    col = lax.broadcasted_iota(
        jnp.int32, (NWIN, HALF), 1).astype(jnp.float32)
    v1, i1 = _half_top4(attn[:, :HALF], col)
    v2, i2 = _half_top4(attn[:, HALF:], col + float(HALF))
    cv_ref[0] = jnp.concatenate(v1 + v2, axis=-1)
    ci_ref[0] = jnp.concatenate(i1 + i2, axis=-1).astype(jnp.int32)


def _tc_candidates(g_win, Wq, bq, Wk, bk):
    B = g_win.shape[0]
    grid_spec = pl.GridSpec(
        grid=(B,),
        in_specs=[
            pl.BlockSpec((1, NWIN, QK), lambda b: (b, 0, 0)),
            pl.BlockSpec((QK, QK), lambda b: (0, 0)),
            pl.BlockSpec((1, QK), lambda b: (0, 0)),
            pl.BlockSpec((QK, QK), lambda b: (0, 0)),
            pl.BlockSpec((1, QK), lambda b: (0, 0)),
        ],
        out_specs=[
            pl.BlockSpec((1, NWIN, 2 * K), lambda b: (b, 0, 0)),
            pl.BlockSpec((1, NWIN, 2 * K), lambda b: (b, 0, 0)),
        ],
    )
    return pl.pallas_call(
        _tc_body,
        grid_spec=grid_spec,
        out_shape=[
            jax.ShapeDtypeStruct((B, NWIN, 2 * K), jnp.float32),
            jax.ShapeDtypeStruct((B, NWIN, 2 * K), jnp.int32),
        ],
    )(g_win, Wq, bq.reshape(1, QK), Wk, bk.reshape(1, QK))


def _take16(x, idx):
    dnums = lax.GatherDimensionNumbers(
        offset_dims=(), collapsed_slice_dims=(0,), start_index_map=(0,))
    return lax.gather(x, idx[:, None], dnums, slice_sizes=(1,),
                      mode=lax.GatherScatterMode.PROMISE_IN_BOUNDS)


def _sort8_desc(v, i, lanes):
    l7 = lanes & 7
    for d in (7, 2, 1):
        p = (lanes & ~7) | (l7 ^ d)
        pv = _take16(v, p)
        pi = _take16(i, p)
        take = (v > pv) | ((v == pv) & (i < pi))
        inv = lanes > p
        keep = take != inv
        v = jnp.where(keep, v, pv)
        i = jnp.where(keep, i, pi)
    return v, i


def _softmax8(v, lanes):
    m = _take16(v, lanes & 8)
    e = jnp.exp(v - m)
    s = e + _take16(e, lanes ^ 1)
    s = s + _take16(s, lanes ^ 2)
    return e / s


def _sc_merge(cv2d, ci2d):
    mesh = plsc.VectorSubcoreMesh(core_axis_name="c", subcore_axis_name="s")
    info = plsc.get_sparse_core_info()
    nc = info.num_cores
    n_in = cv2d.shape[1]
    n_out = n_in // 2

    @functools.partial(
        pl.kernel,
        mesh=mesh,
        out_type=[
            jax.ShapeDtypeStruct((NW, n_out), jnp.float32),
            jax.ShapeDtypeStruct((NW, n_out), jnp.int32),
        ],
        scratch_types=[
            pltpu.VMEM((n_in,), jnp.float32),
            pltpu.VMEM((n_in,), jnp.int32),
            pltpu.VMEM((n_out,), jnp.float32),
            pltpu.VMEM((n_out,), jnp.int32),
        ],
    )
    def sc_kernel(cv_hbm, ci_hbm, w_hbm, ix_hbm, vbuf, ibuf, wbuf, obuf):
        w = lax.axis_index("s") * nc + lax.axis_index("c")
        pltpu.sync_copy(cv_hbm.at[w], vbuf)
        pltpu.sync_copy(ci_hbm.at[w], ibuf)
        lanes = lax.iota(jnp.int32, 16)
        pk = (lanes & 3) + ((lanes & 4) << 1)
        low8 = lanes < 8

        def body(t, _):
            base = t * 128
            out_w = []
            out_i = []
            for hv in range(0, 128, 16):
                v = vbuf[pl.ds(base + hv, 16)]
                i = ibuf[pl.ds(base + hv, 16)]
                v, i = _sort8_desc(v, i, lanes)
                sw = _softmax8(v, lanes)
                out_w.append(_take16(sw, pk))
                out_i.append(_take16(i, pk))
            for g in range(4):
                wbuf[pl.ds(t * 64 + g * 16, 16)] = jnp.where(
                    low8, out_w[2 * g], out_w[2 * g + 1])
                obuf[pl.ds(t * 64 + g * 16, 16)] = jnp.where(
                    low8, out_i[2 * g], out_i[2 * g + 1])
            return 0

        lax.fori_loop(0, n_in // 128, body, 0)
        pltpu.sync_copy(wbuf, w_hbm.at[w])
        pltpu.sync_copy(obuf, ix_hbm.at[w])

    return sc_kernel(cv2d, ci2d)


@jax.jit
def kernel(g_win, Wq, bq, Wk, bk):
    B = g_win.shape[0]
    cv, ci = _tc_candidates(g_win, Wq, bq, Wk, bk)
    w2d, i2d = _sc_merge(cv.reshape(NW, B * NWIN * 2 * K // NW),
                         ci.reshape(NW, B * NWIN * 2 * K // NW))
    return (w2d.reshape(B, NWIN, K), i2d.reshape(B, NWIN, K))
